# Initial kernel scaffold; baseline (speedup 1.0000x reference)
import jax
import jax.numpy as jnp
from jax.experimental import pallas as pl


def kernel(x):
    raise NotImplementedError("write your pallas kernel here")


# speedup vs baseline: 31.4260x; 31.4260x over previous
import functools

import jax
import jax.numpy as jnp
from jax.experimental import pallas as pl

_K_CUT = 3276
_ROWS_PER_BLOCK = 16


def _dropmax_block(x_ref, o_ref):
    x = x_ref[...]
    bits = jax.lax.bitcast_convert_type(x, jnp.int32)
    key = jnp.where(bits < 0, bits ^ jnp.int32(0x7FFFFFFF), bits)

    lo0 = jnp.full((x.shape[0], 1), jnp.iinfo(jnp.int32).min, jnp.int32)
    hi0 = jnp.full((x.shape[0], 1), jnp.iinfo(jnp.int32).max, jnp.int32)

    def body(_, carry):
        lo, hi = carry
        mid = (lo | hi) - ((lo ^ hi) >> 1)
        cnt = jnp.sum((key >= mid).astype(jnp.int32), axis=1, keepdims=True)
        pred = cnt >= _K_CUT
        lo = jnp.where(pred, mid, lo)
        hi = jnp.where(pred, hi, mid - 1)
        return lo, hi

    lo, _ = jax.lax.fori_loop(0, 32, body, (lo0, hi0))
    o_ref[...] = jnp.where(key >= lo, jnp.float32(0.0), x)


@jax.jit
def kernel(x):
    b, n = x.shape
    grid = b // _ROWS_PER_BLOCK
    return pl.pallas_call(
        _dropmax_block,
        grid=(grid,),
        in_specs=[pl.BlockSpec((_ROWS_PER_BLOCK, n), lambda i: (i, 0))],
        out_specs=pl.BlockSpec((_ROWS_PER_BLOCK, n), lambda i: (i, 0)),
        out_shape=jax.ShapeDtypeStruct((b, n), jnp.float32),
    )(x)
